# baseline (speedup 1.0000x reference)
import functools
import math

import jax
import jax.numpy as jnp
from jax import lax
from jax.experimental import pallas as pl
from jax.experimental.pallas import tpu as pltpu
from jax.experimental.pallas import tpu_sc as plsc

B, T, L = 4, 2, 4096
V, D = 100000, 128
EPS = 1e-12

NC, NS = 2, 16
NW = NC * NS
N_ROWS = B * T * L
ROWS_PER_W = N_ROWS // NW
OUT_ROWS_PER_W = (B * L) // NW
CH = 128
NCHUNK = ROWS_PER_W // CH
CPH = NCHUNK // T
NBUF = 6
DEPTH = 3

BLK = 1024
NBLK = (B * L) // BLK
BPB = L // BLK


def _sc_gather_body(hh, n_splits, table_hbm, ids_hbm, out_hbm, idx_v,
                    *bufs_and_sems):
    out_rows = (B * L) // n_splits
    rows_per_w = out_rows // NW
    nchunk = (rows_per_w * T) // CH
    cph = nchunk // T
    bufs = bufs_and_sems[:NBUF]
    gsems = bufs_and_sems[NBUF:2 * NBUF]
    ssems = bufs_and_sems[2 * NBUF:3 * NBUF]
    isem = bufs_and_sems[3 * NBUF]
    wid = lax.axis_index("s") * NC + lax.axis_index("c")
    row_base = wid * rows_per_w
    grow_base = hh * out_rows + wid * rows_per_w
    b_idx = grow_base // L
    l_base = grow_base % L
    ihandles = []
    for cc in range(nchunk):
        h, c = cc // cph, cc % cph
        ihandles.append(pltpu.async_copy(
            ids_hbm.at[b_idx, h, pl.ds(l_base + c * CH, CH)],
            idx_v.at[cc], isem))
    for hnd in ihandles:
        hnd.wait()
    ghandles = [None] * NBUF
    shandles = [None] * NBUF
    for cc in range(nchunk + DEPTH):
        if cc < nchunk:
            bi = cc % NBUF
            if cc >= NBUF:
                shandles[bi].wait()
            ghandles[bi] = pltpu.async_copy(
                table_hbm.at[idx_v.at[cc]], bufs[bi], gsems[bi])
        d = cc - DEPTH
        if 0 <= d < nchunk:
            bj = d % NBUF
            ghandles[bj].wait()
            h, c = d // cph, d % cph
            shandles[bj] = pltpu.async_copy(
                bufs[bj],
                out_hbm.at[pl.ds(row_base + c * CH, CH), pl.ds(h * D, D)],
                ssems[bj])
    for d in range(max(0, nchunk - NBUF), nchunk):
        shandles[d % NBUF].wait()


@functools.cache
def _build_sc_gather(hh, n_splits):
    out_rows = (B * L) // n_splits
    nchunk = (out_rows // NW * T) // CH
    return pl.kernel(
        functools.partial(_sc_gather_body, hh, n_splits),
        mesh=plsc.VectorSubcoreMesh(core_axis_name="c", subcore_axis_name="s"),
        out_type=jax.ShapeDtypeStruct((out_rows, T * D), jnp.float32),
        scratch_types=[
            pltpu.VMEM((nchunk, CH), jnp.int32),
        ] + [pltpu.VMEM((CH, D), jnp.float32) for _ in range(NBUF)]
          + [pltpu.SemaphoreType.DMA for _ in range(2 * NBUF + 1)],
    )


def _norm_body(ts_ref, tt_ref, g_ref, type_ref, w1_ref, b1_ref, w2_ref,
               b2_ref, gamma_ref, beta_ref, out_ref):
    t_val = ts_ref[0]
    col = lax.broadcasted_iota(jnp.int32, (1, D), 1)
    is_cos = col < (D // 2)
    k = jnp.where(is_cos, col, col - D // 2).astype(jnp.float32)
    freqs = jnp.exp((-math.log(10000.0) / (D // 2)) * k)
    args = t_val * freqs
    te = jnp.where(is_cos, jnp.cos(args), jnp.sin(args))
    h = jnp.dot(te, w1_ref[...], preferred_element_type=jnp.float32) + b1_ref[...]
    h = h * jax.nn.sigmoid(h)
    trow = jnp.dot(h, w2_ref[...], preferred_element_type=jnp.float32) + b2_ref[...]

    ty0 = type_ref[0:1, :]
    dty = type_ref[1:2, :] - ty0
    base_row = ty0 + trow
    g = g_ref[...]
    tt = tt_ref[0]
    half0 = g[:, :D] + base_row + tt[:, 0:1] * dty
    half1 = g[:, D:] + base_row + tt[:, 1:2] * dty
    emb = jnp.concatenate([half0, half1], axis=-1)
    mu = jnp.mean(emb, axis=-1, keepdims=True)
    dev = emb - mu
    var = jnp.mean(dev * dev, axis=-1, keepdims=True)
    out_ref[0] = dev * lax.rsqrt(var + EPS) * gamma_ref[...] + beta_ref[...]


NSPLIT = 1
BH = B // NSPLIT
NBLK_H = NBLK // NSPLIT


def _norm_body_alias(alias_ref, *refs):
    del alias_ref
    _norm_body(*refs)


def _make_norm(hh):
    off = hh * NBLK_H
    specs = [
        pl.BlockSpec((1, 1, D), lambda i: ((i + off) // BPB, 0, 0)),
        pl.BlockSpec((1, BLK, T), lambda i: (i + off, 0, 0)),
        pl.BlockSpec((BLK, T * D), lambda i: (i, 0)),
        pl.BlockSpec((2, D), lambda i: (0, 0)),
        pl.BlockSpec((D, D), lambda i: (0, 0)),
        pl.BlockSpec((1, D), lambda i: (0, 0)),
        pl.BlockSpec((D, D), lambda i: (0, 0)),
        pl.BlockSpec((1, D), lambda i: (0, 0)),
        pl.BlockSpec((1, T * D), lambda i: (0, 0)),
        pl.BlockSpec((1, T * D), lambda i: (0, 0)),
    ]
    out_spec = pl.BlockSpec(
        (1, BLK, T * D), lambda i: ((i + off) // BPB, (i + off) % BPB, 0))
    out_shape = jax.ShapeDtypeStruct((B, L, T * D), jnp.float32)
    if hh == 0:
        return pl.pallas_call(_norm_body, grid=(NBLK_H,), in_specs=specs,
                              out_specs=out_spec, out_shape=out_shape)
    return pl.pallas_call(
        _norm_body_alias, grid=(NBLK_H,),
        in_specs=[pl.BlockSpec(memory_space=pl.ANY)] + specs,
        out_specs=out_spec, out_shape=out_shape,
        input_output_aliases={0: 0})


_norms = [_make_norm(hh) for hh in range(NSPLIT)]


def kernel(input_ids, token_type_ids, timesteps, word_emb, type_emb,
           W1, b1, W2, b2, gamma, beta):
    ids = input_ids.astype(jnp.int32)
    tt3 = jnp.transpose(token_type_ids, (0, 2, 1)).astype(jnp.float32).reshape(NBLK, BLK, T)
    ts_b = jnp.broadcast_to(
        timesteps.astype(jnp.float32)[:, None], (B, D)).reshape(B, 1, D)
    w1t, w2t = W1.T, W2.T
    b1r, b2r = b1.reshape(1, D), b2.reshape(1, D)
    gr, br = gamma.reshape(1, T * D), beta.reshape(1, T * D)
    halves = [_build_sc_gather(hh, NSPLIT)(word_emb, ids) for hh in range(NSPLIT)]
    out = _norms[0](ts_b, tt3, halves[0], type_emb, w1t, b1r, w2t, b2r, gr, br)
    for hh in range(1, NSPLIT):
        out = _norms[hh](out, ts_b, tt3, halves[hh], type_emb, w1t, b1r, w2t,
                         b2r, gr, br)
    return out


# speedup vs baseline: 1.0867x; 1.0867x over previous
import functools
import math

import jax
import jax.numpy as jnp
from jax import lax
from jax.experimental import pallas as pl
from jax.experimental.pallas import tpu as pltpu
from jax.experimental.pallas import tpu_sc as plsc

B, T, L = 4, 2, 4096
V, D = 100000, 128
EPS = 1e-12

NC, NS = 2, 16
NW = NC * NS
N_ROWS = B * T * L
ROWS_PER_W = N_ROWS // NW
OUT_ROWS_PER_W = (B * L) // NW
CH = 128
NCHUNK = ROWS_PER_W // CH
CPH = NCHUNK // T
NBUF = 6
DEPTH = 3

BLK = 4096
NBLK = (B * L) // BLK
BPB = L // BLK


def _sc_gather_body(hh, n_splits, table_hbm, ids_hbm, out_hbm, idx_v,
                    *bufs_and_sems):
    out_rows = (B * L) // n_splits
    rows_per_w = out_rows // NW
    nchunk = (rows_per_w * T) // CH
    cph = nchunk // T
    bufs = bufs_and_sems[:NBUF]
    gsems = bufs_and_sems[NBUF:2 * NBUF]
    ssems = bufs_and_sems[2 * NBUF:3 * NBUF]
    isem = bufs_and_sems[3 * NBUF]
    wid = lax.axis_index("s") * NC + lax.axis_index("c")
    row_base = wid * rows_per_w
    grow_base = hh * out_rows + wid * rows_per_w
    b_idx = grow_base // L
    l_base = grow_base % L
    ihandles = []
    for cc in range(nchunk):
        h, c = cc // cph, cc % cph
        ihandles.append(pltpu.async_copy(
            ids_hbm.at[b_idx, h, pl.ds(l_base + c * CH, CH)],
            idx_v.at[cc], isem))
    for hnd in ihandles:
        hnd.wait()
    ghandles = [None] * NBUF
    shandles = [None] * NBUF
    for cc in range(nchunk + DEPTH):
        if cc < nchunk:
            bi = cc % NBUF
            if cc >= NBUF:
                shandles[bi].wait()
            ghandles[bi] = pltpu.async_copy(
                table_hbm.at[idx_v.at[cc]], bufs[bi], gsems[bi])
        d = cc - DEPTH
        if 0 <= d < nchunk:
            bj = d % NBUF
            ghandles[bj].wait()
            h, c = d // cph, d % cph
            shandles[bj] = pltpu.async_copy(
                bufs[bj],
                out_hbm.at[pl.ds(row_base + c * CH, CH), pl.ds(h * D, D)],
                ssems[bj])
    for d in range(max(0, nchunk - NBUF), nchunk):
        shandles[d % NBUF].wait()


@functools.cache
def _build_sc_gather(hh, n_splits):
    out_rows = (B * L) // n_splits
    nchunk = (out_rows // NW * T) // CH
    return pl.kernel(
        functools.partial(_sc_gather_body, hh, n_splits),
        mesh=plsc.VectorSubcoreMesh(core_axis_name="c", subcore_axis_name="s"),
        out_type=jax.ShapeDtypeStruct((out_rows, T * D), jnp.float32),
        scratch_types=[
            pltpu.VMEM((nchunk, CH), jnp.int32),
        ] + [pltpu.VMEM((CH, D), jnp.float32) for _ in range(NBUF)]
          + [pltpu.SemaphoreType.DMA for _ in range(2 * NBUF + 1)],
    )


def _norm_body(ts_ref, tt_ref, g_ref, type_ref, w1_ref, b1_ref, w2_ref,
               b2_ref, gamma_ref, beta_ref, out_ref):
    t_val = ts_ref[0]
    col = lax.broadcasted_iota(jnp.int32, (1, D), 1)
    is_cos = col < (D // 2)
    k = jnp.where(is_cos, col, col - D // 2).astype(jnp.float32)
    freqs = jnp.exp((-math.log(10000.0) / (D // 2)) * k)
    args = t_val * freqs
    te = jnp.where(is_cos, jnp.cos(args), jnp.sin(args))
    h = jnp.dot(te, w1_ref[...], preferred_element_type=jnp.float32) + b1_ref[...]
    h = h * jax.nn.sigmoid(h)
    trow = jnp.dot(h, w2_ref[...], preferred_element_type=jnp.float32) + b2_ref[...]

    ty0 = type_ref[0:1, :]
    dty = type_ref[1:2, :] - ty0
    base_row = ty0 + trow
    g = g_ref[...]
    tt = tt_ref[0]
    half0 = g[:, :D] + base_row + tt[:, 0:1] * dty
    half1 = g[:, D:] + base_row + tt[:, 1:2] * dty
    emb = jnp.concatenate([half0, half1], axis=-1)
    mu = jnp.mean(emb, axis=-1, keepdims=True)
    dev = emb - mu
    var = jnp.mean(dev * dev, axis=-1, keepdims=True)
    out_ref[0] = dev * lax.rsqrt(var + EPS) * gamma_ref[...] + beta_ref[...]


NSPLIT = 1
BH = B // NSPLIT
NBLK_H = NBLK // NSPLIT


def _norm_body_alias(alias_ref, *refs):
    del alias_ref
    _norm_body(*refs)


def _make_norm(hh):
    off = hh * NBLK_H
    specs = [
        pl.BlockSpec((1, 1, D), lambda i: ((i + off) // BPB, 0, 0)),
        pl.BlockSpec((1, BLK, T), lambda i: (i + off, 0, 0)),
        pl.BlockSpec((BLK, T * D), lambda i: (i, 0)),
        pl.BlockSpec((2, D), lambda i: (0, 0)),
        pl.BlockSpec((D, D), lambda i: (0, 0)),
        pl.BlockSpec((1, D), lambda i: (0, 0)),
        pl.BlockSpec((D, D), lambda i: (0, 0)),
        pl.BlockSpec((1, D), lambda i: (0, 0)),
        pl.BlockSpec((1, T * D), lambda i: (0, 0)),
        pl.BlockSpec((1, T * D), lambda i: (0, 0)),
    ]
    out_spec = pl.BlockSpec(
        (1, BLK, T * D), lambda i: ((i + off) // BPB, (i + off) % BPB, 0))
    out_shape = jax.ShapeDtypeStruct((B, L, T * D), jnp.float32)
    if hh == 0:
        return pl.pallas_call(_norm_body, grid=(NBLK_H,), in_specs=specs,
                              out_specs=out_spec, out_shape=out_shape)
    return pl.pallas_call(
        _norm_body_alias, grid=(NBLK_H,),
        in_specs=[pl.BlockSpec(memory_space=pl.ANY)] + specs,
        out_specs=out_spec, out_shape=out_shape,
        input_output_aliases={0: 0})


_norms = [_make_norm(hh) for hh in range(NSPLIT)]


def kernel(input_ids, token_type_ids, timesteps, word_emb, type_emb,
           W1, b1, W2, b2, gamma, beta):
    ids = input_ids.astype(jnp.int32)
    tt3 = jnp.transpose(token_type_ids, (0, 2, 1)).astype(jnp.float32).reshape(NBLK, BLK, T)
    ts_b = jnp.broadcast_to(
        timesteps.astype(jnp.float32)[:, None], (B, D)).reshape(B, 1, D)
    w1t, w2t = W1.T, W2.T
    b1r, b2r = b1.reshape(1, D), b2.reshape(1, D)
    gr, br = gamma.reshape(1, T * D), beta.reshape(1, T * D)
    halves = [_build_sc_gather(hh, NSPLIT)(word_emb, ids) for hh in range(NSPLIT)]
    out = _norms[0](ts_b, tt3, halves[0], type_emb, w1t, b1r, w2t, b2r, gr, br)
    for hh in range(1, NSPLIT):
        out = _norms[hh](out, ts_b, tt3, halves[hh], type_emb, w1t, b1r, w2t,
                         b2r, gr, br)
    return out


# speedup vs baseline: 1.0885x; 1.0017x over previous
import functools
import math

import jax
import jax.numpy as jnp
from jax import lax
from jax.experimental import pallas as pl
from jax.experimental.pallas import tpu as pltpu
from jax.experimental.pallas import tpu_sc as plsc

B, T, L = 4, 2, 4096
V, D = 100000, 128
EPS = 1e-12

NC, NS = 2, 16
NW = NC * NS
N_ROWS = B * T * L
ROWS_PER_W = N_ROWS // NW
OUT_ROWS_PER_W = (B * L) // NW
CH = 128
NCHUNK = ROWS_PER_W // CH
CPH = NCHUNK // T
NBUF = 6
DEPTH = 3

BLK = 4096
NBLK = (B * L) // BLK
BPB = L // BLK


def _sc_gather_body(hh, n_splits, table_hbm, ids_hbm, out_hbm, idx_v,
                    *bufs_and_sems):
    out_rows = (B * L) // n_splits
    rows_per_w = out_rows // NW
    nchunk = (rows_per_w * T) // CH
    cph = nchunk // T
    bufs = bufs_and_sems[:NBUF]
    gsems = bufs_and_sems[NBUF:2 * NBUF]
    ssems = bufs_and_sems[2 * NBUF:3 * NBUF]
    isem = bufs_and_sems[3 * NBUF]
    wid = lax.axis_index("s") * NC + lax.axis_index("c")
    row_base = wid * rows_per_w
    grow_base = hh * out_rows + wid * rows_per_w
    b_idx = grow_base // L
    l_base = grow_base % L
    ihandles = []
    for cc in range(nchunk):
        h, c = cc // cph, cc % cph
        ihandles.append(pltpu.async_copy(
            ids_hbm.at[b_idx, h, pl.ds(l_base + c * CH, CH)],
            idx_v.at[cc], isem))
    for hnd in ihandles:
        hnd.wait()
    ghandles = [None] * NBUF
    shandles = [None] * NBUF
    for cc in range(nchunk + DEPTH):
        if cc < nchunk:
            bi = cc % NBUF
            if cc >= NBUF:
                shandles[bi].wait()
            ghandles[bi] = pltpu.async_copy(
                table_hbm.at[idx_v.at[cc]], bufs[bi], gsems[bi])
        d = cc - DEPTH
        if 0 <= d < nchunk:
            bj = d % NBUF
            ghandles[bj].wait()
            h, c = d // cph, d % cph
            shandles[bj] = pltpu.async_copy(
                bufs[bj],
                out_hbm.at[pl.ds(row_base + c * CH, CH), pl.ds(h * D, D)],
                ssems[bj])
    for d in range(max(0, nchunk - NBUF), nchunk):
        shandles[d % NBUF].wait()


@functools.cache
def _build_sc_gather(hh, n_splits):
    out_rows = (B * L) // n_splits
    nchunk = (out_rows // NW * T) // CH
    return pl.kernel(
        functools.partial(_sc_gather_body, hh, n_splits),
        mesh=plsc.VectorSubcoreMesh(core_axis_name="c", subcore_axis_name="s"),
        out_type=jax.ShapeDtypeStruct((out_rows, T * D), jnp.float32),
        scratch_types=[
            pltpu.VMEM((nchunk, CH), jnp.int32),
        ] + [pltpu.VMEM((CH, D), jnp.float32) for _ in range(NBUF)]
          + [pltpu.SemaphoreType.DMA for _ in range(2 * NBUF + 1)],
    )


def _norm_body(ts_ref, tt_ref, g_ref, type_ref, w1_ref, b1_ref, w2_ref,
               b2_ref, gamma_ref, beta_ref, out_ref, scr_ref):
    i = pl.program_id(0)

    @pl.when(i == 0)
    def _mlp():
        t_all = ts_ref[:, 0, :]
        col = lax.broadcasted_iota(jnp.int32, (B, D), 1)
        is_cos = col < (D // 2)
        k = jnp.where(is_cos, col, col - D // 2).astype(jnp.float32)
        freqs = jnp.exp((-math.log(10000.0) / (D // 2)) * k)
        args = t_all * freqs
        te = jnp.where(is_cos, jnp.cos(args), jnp.sin(args))
        h = jnp.dot(te, w1_ref[...], preferred_element_type=jnp.float32) + b1_ref[...]
        h = h * jax.nn.sigmoid(h)
        trow = jnp.dot(h, w2_ref[...], preferred_element_type=jnp.float32) + b2_ref[...]
        scr_ref[...] = trow + type_ref[0:1, :]

    b = i // BPB
    base_row = scr_ref[pl.ds(b, 1), :]
    dty = type_ref[1:2, :] - type_ref[0:1, :]
    g = g_ref[...]
    tt = tt_ref[0]
    half0 = g[:, :D] + base_row + tt[:, 0:1] * dty
    half1 = g[:, D:] + base_row + tt[:, 1:2] * dty
    emb = jnp.concatenate([half0, half1], axis=-1)
    mu = jnp.mean(emb, axis=-1, keepdims=True)
    dev = emb - mu
    var = jnp.mean(dev * dev, axis=-1, keepdims=True)
    out_ref[0] = dev * lax.rsqrt(var + EPS) * gamma_ref[...] + beta_ref[...]


NSPLIT = 1
BH = B // NSPLIT
NBLK_H = NBLK // NSPLIT


def _norm_body_alias(alias_ref, *refs):
    del alias_ref
    _norm_body(*refs)


def _make_norm(hh):
    off = hh * NBLK_H
    specs = [
        pl.BlockSpec((B, 1, D), lambda i: (0, 0, 0)),
        pl.BlockSpec((1, BLK, T), lambda i: (i + off, 0, 0)),
        pl.BlockSpec((BLK, T * D), lambda i: (i, 0)),
        pl.BlockSpec((2, D), lambda i: (0, 0)),
        pl.BlockSpec((D, D), lambda i: (0, 0)),
        pl.BlockSpec((1, D), lambda i: (0, 0)),
        pl.BlockSpec((D, D), lambda i: (0, 0)),
        pl.BlockSpec((1, D), lambda i: (0, 0)),
        pl.BlockSpec((1, T * D), lambda i: (0, 0)),
        pl.BlockSpec((1, T * D), lambda i: (0, 0)),
    ]
    out_spec = pl.BlockSpec(
        (1, BLK, T * D), lambda i: ((i + off) // BPB, (i + off) % BPB, 0))
    out_shape = jax.ShapeDtypeStruct((B, L, T * D), jnp.float32)
    scratch = [pltpu.VMEM((B, D), jnp.float32)]
    if hh == 0:
        return pl.pallas_call(_norm_body, grid=(NBLK_H,), in_specs=specs,
                              out_specs=out_spec, out_shape=out_shape,
                              scratch_shapes=scratch)
    return pl.pallas_call(
        _norm_body_alias, grid=(NBLK_H,),
        in_specs=[pl.BlockSpec(memory_space=pl.ANY)] + specs,
        out_specs=out_spec, out_shape=out_shape,
        input_output_aliases={0: 0}, scratch_shapes=scratch)


_norms = [_make_norm(hh) for hh in range(NSPLIT)]


def kernel(input_ids, token_type_ids, timesteps, word_emb, type_emb,
           W1, b1, W2, b2, gamma, beta):
    ids = input_ids.astype(jnp.int32)
    tt3 = jnp.transpose(token_type_ids, (0, 2, 1)).astype(jnp.float32).reshape(NBLK, BLK, T)
    ts_b = jnp.broadcast_to(
        timesteps.astype(jnp.float32)[:, None], (B, D)).reshape(B, 1, D)
    w1t, w2t = W1.T, W2.T
    b1r, b2r = b1.reshape(1, D), b2.reshape(1, D)
    gr, br = gamma.reshape(1, T * D), beta.reshape(1, T * D)
    halves = [_build_sc_gather(hh, NSPLIT)(word_emb, ids) for hh in range(NSPLIT)]
    out = _norms[0](ts_b, tt3, halves[0], type_emb, w1t, b1r, w2t, b2r, gr, br)
    for hh in range(1, NSPLIT):
        out = _norms[hh](out, ts_b, tt3, halves[hh], type_emb, w1t, b1r, w2t,
                         b2r, gr, br)
    return out


# speedup vs baseline: 1.0921x; 1.0034x over previous
import functools
import math

import jax
import jax.numpy as jnp
from jax import lax
from jax.experimental import pallas as pl
from jax.experimental.pallas import tpu as pltpu
from jax.experimental.pallas import tpu_sc as plsc

B, T, L = 4, 2, 4096
V, D = 100000, 128
EPS = 1e-12

NC, NS = 2, 16
NW = NC * NS
N_ROWS = B * T * L
ROWS_PER_W = N_ROWS // NW
OUT_ROWS_PER_W = (B * L) // NW
CH = 128
NCHUNK = ROWS_PER_W // CH
CPH = NCHUNK // T
NBUF = 6
DEPTH = 3

BLK = 4096
NBLK = (B * L) // BLK
BPB = L // BLK


def _sc_gather_body(hh, n_splits, table_hbm, ids_hbm, out_hbm, idx_v,
                    *bufs_and_sems):
    out_rows = (B * L) // n_splits
    rows_per_w = out_rows // NW
    nchunk = (rows_per_w * T) // CH
    cph = nchunk // T
    bufs = bufs_and_sems[:NBUF]
    gsems = bufs_and_sems[NBUF:2 * NBUF]
    ssems = bufs_and_sems[2 * NBUF:3 * NBUF]
    isem = bufs_and_sems[3 * NBUF]
    wid = lax.axis_index("s") * NC + lax.axis_index("c")
    row_base = wid * rows_per_w
    grow_base = hh * out_rows + wid * rows_per_w
    b_idx = grow_base // L
    l_base = grow_base % L
    ihandles = []
    for cc in range(nchunk):
        h, c = cc // cph, cc % cph
        ihandles.append(pltpu.async_copy(
            ids_hbm.at[b_idx, h, pl.ds(l_base + c * CH, CH)],
            idx_v.at[cc], isem))
    for hnd in ihandles:
        hnd.wait()
    ghandles = [None] * NBUF
    shandles = [None] * NBUF
    for cc in range(nchunk + DEPTH):
        if cc < nchunk:
            bi = cc % NBUF
            if cc >= NBUF:
                shandles[bi].wait()
            ghandles[bi] = pltpu.async_copy(
                table_hbm.at[idx_v.at[cc]], bufs[bi], gsems[bi])
        d = cc - DEPTH
        if 0 <= d < nchunk:
            bj = d % NBUF
            ghandles[bj].wait()
            h, c = d // cph, d % cph
            shandles[bj] = pltpu.async_copy(
                bufs[bj],
                out_hbm.at[pl.ds(row_base + c * CH, CH), pl.ds(h * D, D)],
                ssems[bj])
    for d in range(max(0, nchunk - NBUF), nchunk):
        shandles[d % NBUF].wait()


@functools.cache
def _build_sc_gather(hh, n_splits):
    out_rows = (B * L) // n_splits
    nchunk = (out_rows // NW * T) // CH
    return pl.kernel(
        functools.partial(_sc_gather_body, hh, n_splits),
        mesh=plsc.VectorSubcoreMesh(core_axis_name="c", subcore_axis_name="s"),
        out_type=jax.ShapeDtypeStruct((out_rows, T * D), jnp.float32),
        scratch_types=[
            pltpu.VMEM((nchunk, CH), jnp.int32),
        ] + [pltpu.VMEM((CH, D), jnp.float32) for _ in range(NBUF)]
          + [pltpu.SemaphoreType.DMA for _ in range(2 * NBUF + 1)],
    )


def _norm_body(ts_ref, tt_ref, g_ref, type_ref, w1_ref, b1_ref, w2_ref,
               b2_ref, gamma_ref, beta_ref, out_ref, scr_ref):
    i = pl.program_id(0)

    @pl.when(i == 0)
    def _mlp():
        t_all = ts_ref[:, 0, :]
        col = lax.broadcasted_iota(jnp.int32, (B, D), 1)
        is_cos = col < (D // 2)
        k = jnp.where(is_cos, col, col - D // 2).astype(jnp.float32)
        freqs = jnp.exp((-math.log(10000.0) / (D // 2)) * k)
        args = t_all * freqs
        te = jnp.where(is_cos, jnp.cos(args), jnp.sin(args))
        h = jnp.dot(te, w1_ref[...], preferred_element_type=jnp.float32) + b1_ref[...]
        h = h * jax.nn.sigmoid(h)
        trow = jnp.dot(h, w2_ref[...], preferred_element_type=jnp.float32) + b2_ref[...]
        scr_ref[...] = trow + type_ref[0:1, :]

    b = i // BPB
    base_row = scr_ref[pl.ds(b, 1), :]
    dty = type_ref[1:2, :] - type_ref[0:1, :]
    g = g_ref[...]
    tt = tt_ref[0]
    half0 = g[:, :D] + base_row + tt[:, 0:1] * dty
    half1 = g[:, D:] + base_row + tt[:, 1:2] * dty
    emb = jnp.concatenate([half0, half1], axis=-1)
    mu = jnp.mean(emb, axis=-1, keepdims=True)
    dev = emb - mu
    var = jnp.mean(dev * dev, axis=-1, keepdims=True)
    out_ref[0] = dev * lax.rsqrt(var + EPS) * gamma_ref[...] + beta_ref[...]


_norm = pl.pallas_call(
    _norm_body,
    grid=(NBLK,),
    in_specs=[
        pl.BlockSpec((B, 1, D), lambda i: (0, 0, 0)),
        pl.BlockSpec((1, BLK, T), lambda i: (i, 0, 0)),
        pl.BlockSpec((BLK, T * D), lambda i: (i, 0)),
        pl.BlockSpec((2, D), lambda i: (0, 0)),
        pl.BlockSpec((D, D), lambda i: (0, 0)),
        pl.BlockSpec((1, D), lambda i: (0, 0)),
        pl.BlockSpec((D, D), lambda i: (0, 0)),
        pl.BlockSpec((1, D), lambda i: (0, 0)),
        pl.BlockSpec((1, T * D), lambda i: (0, 0)),
        pl.BlockSpec((1, T * D), lambda i: (0, 0)),
    ],
    out_specs=pl.BlockSpec((1, BLK, T * D), lambda i: (i // BPB, i % BPB, 0)),
    out_shape=jax.ShapeDtypeStruct((B, L, T * D), jnp.float32),
    scratch_shapes=[pltpu.VMEM((B, D), jnp.float32)],
)


def kernel(input_ids, token_type_ids, timesteps, word_emb, type_emb,
           W1, b1, W2, b2, gamma, beta):
    ids = input_ids.astype(jnp.int32)
    tt3 = jnp.transpose(token_type_ids, (0, 2, 1)).astype(jnp.float32).reshape(NBLK, BLK, T)
    ts_b = jnp.broadcast_to(
        timesteps.astype(jnp.float32)[:, None], (B, D)).reshape(B, 1, D)
    gathered = _build_sc_gather(0, 1)(word_emb, ids)
    return _norm(ts_b, tt3, gathered, type_emb, W1.T, b1.reshape(1, D),
                 W2.T, b2.reshape(1, D), gamma.reshape(1, T * D),
                 beta.reshape(1, T * D))
